# baseline (speedup 1.0000x reference)
import jax
import jax.numpy as jnp
from jax.experimental import pallas as pl
import jax.experimental.pallas.tpu as pltpu

B, T, C = 4, 2048, 1024
E = 64
K = 2
BT = B * T
BLK = 512


def _outer(x_hbm, w_ref, b_ref, probs_hbm, topk_hbm, idx_hbm):
    def _inner(x_ref, probs_ref, topk_ref, idx_ref):
        g = jnp.dot(x_ref[...], w_ref[...], preferred_element_type=jnp.float32)
        g = g + b_ref[...]
        m = jnp.max(g, axis=-1, keepdims=True)
        e = jnp.exp(g - m)
        s = jnp.sum(e, axis=-1, keepdims=True)
        r = 1.0 / s
        probs_ref[...] = e * r

        lanesf = jax.lax.broadcasted_iota(jnp.int32, e.shape, 1).astype(jnp.float32)
        i1f = jnp.min(jnp.where(e == 1.0, lanesf, float(E)), axis=-1, keepdims=True)
        e2 = jnp.where(lanesf == i1f, -1.0, e)
        m2 = jnp.max(e2, axis=-1, keepdims=True)
        i2f = jnp.min(jnp.where(e2 == m2, lanesf, float(E)), axis=-1, keepdims=True)
        topk_ref[...] = jnp.concatenate([r, m2 * r], axis=-1)
        idx_ref[...] = jnp.concatenate([i1f, i2f], axis=-1).astype(jnp.int32)

    pltpu.emit_pipeline(
        _inner,
        grid=(BT // BLK,),
        in_specs=[pl.BlockSpec((BLK, C), lambda i: (i, 0))],
        out_specs=[
            pl.BlockSpec((BLK, E), lambda i: (i, 0)),
            pl.BlockSpec((BLK, K), lambda i: (i, 0)),
            pl.BlockSpec((BLK, K), lambda i: (i, 0)),
        ],
    )(x_hbm, probs_hbm, topk_hbm, idx_hbm)


@jax.jit
def kernel(x, W_router, b_router):
    x2 = x.reshape(BT, C)
    b2 = b_router.reshape(1, E)
    probs, topk, idx = pl.pallas_call(
        _outer,
        in_specs=[
            pl.BlockSpec(memory_space=pl.ANY),
            pl.BlockSpec(memory_space=pltpu.VMEM),
            pl.BlockSpec(memory_space=pltpu.VMEM),
        ],
        out_specs=[
            pl.BlockSpec(memory_space=pl.ANY),
            pl.BlockSpec(memory_space=pl.ANY),
            pl.BlockSpec(memory_space=pl.ANY),
        ],
        out_shape=[
            jax.ShapeDtypeStruct((BT, E), jnp.float32),
            jax.ShapeDtypeStruct((BT, K), jnp.float32),
            jax.ShapeDtypeStruct((BT, K), jnp.int32),
        ],
    )(x2, W_router, b2)
    return (probs.reshape(B, T, E),
            topk.reshape(B, T, K),
            idx.reshape(B, T, K))


# speedup vs baseline: 1.1361x; 1.1361x over previous
import jax
import jax.numpy as jnp
from jax.experimental import pallas as pl
import jax.experimental.pallas.tpu as pltpu

B, T, C = 4, 2048, 1024
E = 64
K = 2
BT = B * T
BLK = 1024


def _outer(x_hbm, w_ref, b_ref, probs_hbm, topk_hbm, idx_hbm):
    def _inner(x_ref, probs_ref, topk_ref, idx_ref):
        g = jnp.dot(x_ref[...], w_ref[...], preferred_element_type=jnp.float32)
        g = g + b_ref[...]
        m = jnp.max(g, axis=-1, keepdims=True)
        e = jnp.exp(g - m)
        s = jnp.sum(e, axis=-1, keepdims=True)
        r = 1.0 / s
        probs_ref[...] = e * r

        lanesf = jax.lax.broadcasted_iota(jnp.int32, e.shape, 1).astype(jnp.float32)
        i1f = jnp.min(jnp.where(e == 1.0, lanesf, float(E)), axis=-1, keepdims=True)
        e2 = jnp.where(lanesf == i1f, -1.0, e)
        m2 = jnp.max(e2, axis=-1, keepdims=True)
        i2f = jnp.min(jnp.where(e2 == m2, lanesf, float(E)), axis=-1, keepdims=True)
        topk_ref[...] = jnp.concatenate([r, m2 * r], axis=-1)
        idx_ref[...] = jnp.concatenate([i1f, i2f], axis=-1).astype(jnp.int32)

    pltpu.emit_pipeline(
        _inner,
        grid=(BT // BLK,),
        in_specs=[pl.BlockSpec((BLK, C), lambda i: (i, 0))],
        out_specs=[
            pl.BlockSpec((BLK, E), lambda i: (i, 0)),
            pl.BlockSpec((BLK, K), lambda i: (i, 0)),
            pl.BlockSpec((BLK, K), lambda i: (i, 0)),
        ],
    )(x_hbm, probs_hbm, topk_hbm, idx_hbm)


@jax.jit
def kernel(x, W_router, b_router):
    x2 = x.reshape(BT, C)
    b2 = b_router.reshape(1, E)
    probs, topk, idx = pl.pallas_call(
        _outer,
        in_specs=[
            pl.BlockSpec(memory_space=pl.ANY),
            pl.BlockSpec(memory_space=pltpu.VMEM),
            pl.BlockSpec(memory_space=pltpu.VMEM),
        ],
        out_specs=[
            pl.BlockSpec(memory_space=pl.ANY),
            pl.BlockSpec(memory_space=pl.ANY),
            pl.BlockSpec(memory_space=pl.ANY),
        ],
        out_shape=[
            jax.ShapeDtypeStruct((BT, E), jnp.float32),
            jax.ShapeDtypeStruct((BT, K), jnp.float32),
            jax.ShapeDtypeStruct((BT, K), jnp.int32),
        ],
    )(x2, W_router, b2)
    return (probs.reshape(B, T, E),
            topk.reshape(B, T, K),
            idx.reshape(B, T, K))


# speedup vs baseline: 1.1878x; 1.0455x over previous
import jax
import jax.numpy as jnp
from jax.experimental import pallas as pl
import jax.experimental.pallas.tpu as pltpu

B, T, C = 4, 2048, 1024
E = 64
K = 2
BT = B * T
BLK = 2048


def _outer(x_hbm, w_ref, b_ref, probs_hbm, topk_hbm, idx_hbm):
    def _inner(x_ref, probs_ref, topk_ref, idx_ref):
        g = jnp.dot(x_ref[...], w_ref[...], preferred_element_type=jnp.float32)
        g = g + b_ref[...]
        m = jnp.max(g, axis=-1, keepdims=True)
        e = jnp.exp(g - m)
        s = jnp.sum(e, axis=-1, keepdims=True)
        r = 1.0 / s
        probs_ref[...] = e * r

        lanesf = jax.lax.broadcasted_iota(jnp.int32, e.shape, 1).astype(jnp.float32)
        i1f = jnp.min(jnp.where(e == 1.0, lanesf, float(E)), axis=-1, keepdims=True)
        e2 = jnp.where(lanesf == i1f, -1.0, e)
        m2 = jnp.max(e2, axis=-1, keepdims=True)
        i2f = jnp.min(jnp.where(e2 == m2, lanesf, float(E)), axis=-1, keepdims=True)
        topk_ref[...] = jnp.concatenate([r, m2 * r], axis=-1)
        idx_ref[...] = jnp.concatenate([i1f, i2f], axis=-1).astype(jnp.int32)

    pltpu.emit_pipeline(
        _inner,
        grid=(BT // BLK,),
        in_specs=[pl.BlockSpec((BLK, C), lambda i: (i, 0))],
        out_specs=[
            pl.BlockSpec((BLK, E), lambda i: (i, 0)),
            pl.BlockSpec((BLK, K), lambda i: (i, 0)),
            pl.BlockSpec((BLK, K), lambda i: (i, 0)),
        ],
    )(x_hbm, probs_hbm, topk_hbm, idx_hbm)


@jax.jit
def kernel(x, W_router, b_router):
    x2 = x.reshape(BT, C)
    b2 = b_router.reshape(1, E)
    probs, topk, idx = pl.pallas_call(
        _outer,
        in_specs=[
            pl.BlockSpec(memory_space=pl.ANY),
            pl.BlockSpec(memory_space=pltpu.VMEM),
            pl.BlockSpec(memory_space=pltpu.VMEM),
        ],
        out_specs=[
            pl.BlockSpec(memory_space=pl.ANY),
            pl.BlockSpec(memory_space=pl.ANY),
            pl.BlockSpec(memory_space=pl.ANY),
        ],
        out_shape=[
            jax.ShapeDtypeStruct((BT, E), jnp.float32),
            jax.ShapeDtypeStruct((BT, K), jnp.float32),
            jax.ShapeDtypeStruct((BT, K), jnp.int32),
        ],
    )(x2, W_router, b2)
    return (probs.reshape(B, T, E),
            topk.reshape(B, T, K),
            idx.reshape(B, T, K))


# speedup vs baseline: 1.2032x; 1.0130x over previous
import jax
import jax.numpy as jnp
from jax.experimental import pallas as pl
import jax.experimental.pallas.tpu as pltpu

B, T, C = 4, 2048, 1024
E = 64
K = 2
BT = B * T
BLK = 2048


def _outer(x_hbm, w_ref, b_ref, probs_hbm, topk_hbm, idx_hbm):
    def _inner(x_ref, probs_ref, topk_ref, idx_ref):
        g = jnp.dot(x_ref[...], w_ref[...], preferred_element_type=jnp.float32)
        g = g + b_ref[...]
        m = jnp.max(g, axis=-1, keepdims=True)
        e = jnp.exp(g - m)
        s = jnp.sum(e, axis=-1, keepdims=True)
        r = 1.0 / s
        probs_ref[...] = e * r

        lanes = jax.lax.broadcasted_iota(jnp.int32, g.shape, 1)
        i1 = jnp.argmax(g, axis=-1, keepdims=True)
        g2 = jnp.where(lanes == i1, -jnp.inf, g)
        m2v = jnp.max(g2, axis=-1, keepdims=True)
        i2 = jnp.argmax(g2, axis=-1, keepdims=True)
        topk_ref[...] = jnp.concatenate([r, jnp.exp(m2v - m) * r], axis=-1)
        idx_ref[...] = jnp.concatenate([i1, i2], axis=-1)

    pltpu.emit_pipeline(
        _inner,
        grid=(BT // BLK,),
        in_specs=[pl.BlockSpec((BLK, C), lambda i: (i, 0))],
        out_specs=[
            pl.BlockSpec((BLK, E), lambda i: (i, 0)),
            pl.BlockSpec((BLK, K), lambda i: (i, 0)),
            pl.BlockSpec((BLK, K), lambda i: (i, 0)),
        ],
    )(x_hbm, probs_hbm, topk_hbm, idx_hbm)


@jax.jit
def kernel(x, W_router, b_router):
    x2 = x.reshape(BT, C)
    b2 = b_router.reshape(1, E)
    probs, topk, idx = pl.pallas_call(
        _outer,
        in_specs=[
            pl.BlockSpec(memory_space=pl.ANY),
            pl.BlockSpec(memory_space=pltpu.VMEM),
            pl.BlockSpec(memory_space=pltpu.VMEM),
        ],
        out_specs=[
            pl.BlockSpec(memory_space=pl.ANY),
            pl.BlockSpec(memory_space=pl.ANY),
            pl.BlockSpec(memory_space=pl.ANY),
        ],
        out_shape=[
            jax.ShapeDtypeStruct((BT, E), jnp.float32),
            jax.ShapeDtypeStruct((BT, K), jnp.float32),
            jax.ShapeDtypeStruct((BT, K), jnp.int32),
        ],
    )(x2, W_router, b2)
    return (probs.reshape(B, T, E),
            topk.reshape(B, T, K),
            idx.reshape(B, T, K))
